# Initial kernel scaffold; baseline (speedup 1.0000x reference)
import jax
import jax.numpy as jnp
from jax.experimental import pallas as pl


def kernel(h, edge_index, edge_weight, W, b, prelu_a, fc_W, fc_b, att):
    raise NotImplementedError("write your pallas kernel here")


# speedup vs baseline: 3.8442x; 3.8442x over previous
import functools

import jax
import jax.numpy as jnp
from jax import lax
from jax.experimental import pallas as pl
from jax.experimental.pallas import tpu as pltpu
from jax.experimental.pallas import tpu_sc as plsc

NC = 2
NS = 16
LN = 16


def _sc_aggregate(h, src, dst, w):
  N, H = h.shape
  P, E = src.shape
  src = src.reshape(P * E)
  dst = dst.reshape(P * E)
  w = w.reshape(P * E)
  NW = NC * NS
  per_w = E // NW
  CH = 128
  n_chunks = per_w // CH
  tail = per_w - n_chunks * CH
  RA = (N // NS) // 8 * 8
  RB = N - RA * (NS - 1)
  assert RB % 8 == 0 and RB <= 2 * RA
  mesh = plsc.VectorSubcoreMesh(core_axis_name="c", subcore_axis_name="s",
                                num_cores=NC, num_subcores=NS)

  def _scale_rows(rows_ref, w_ref, k):
    def body(g, carry):
      wv = w_ref[pl.ds(g * LN, LN)]
      for j in range(LN):
        wi = wv[j]
        row = g * LN + j
        for c in range(H // LN):
          sl = pl.ds(c * LN, LN)
          rows_ref[row, sl] = rows_ref[row, sl] * wi
      return carry
    lax.fori_loop(0, k // LN, body, 0)

  @functools.partial(
      pl.kernel,
      out_type=jax.ShapeDtypeStruct((P, NC, N, H), jnp.float32),
      mesh=mesh,
      scratch_types=[
          pltpu.VMEM_SHARED((N, H), jnp.float32),
          pltpu.VMEM((CH,), jnp.int32),
          pltpu.VMEM((CH,), jnp.int32),
          pltpu.VMEM((CH,), jnp.float32),
          pltpu.VMEM((CH, H), jnp.float32),
          pltpu.VMEM((LN,), jnp.int32),
          pltpu.VMEM((LN,), jnp.int32),
          pltpu.VMEM((LN,), jnp.float32),
          pltpu.VMEM((LN, H), jnp.float32),
          pltpu.SemaphoreType.DMA,
      ],
  )
  def body(h_hbm, src_hbm, dst_hbm, w_hbm, zero_hbm, parts_hbm,
           acc, src_v, dst_v, w_v, rows_v, src_t, dst_t, w_t, rows_t, sem):
    cid = lax.axis_index("c")
    sid = lax.axis_index("s")
    wid = sid * NC + cid
    for p in range(P):
      base = p * E + wid * per_w
      @pl.when(sid < NS - 1)
      def _():
        pltpu.sync_copy(zero_hbm.at[pl.ds(0, RA)],
                        acc.at[pl.ds(sid * RA, RA)])

      @pl.when(sid == NS - 1)
      def _():
        pltpu.sync_copy(zero_hbm, acc.at[pl.ds((NS - 1) * RA, RB)])

      plsc.subcore_barrier()

      def chunk(i, carry):
        off = base + i * CH
        pltpu.sync_copy(src_hbm.at[pl.ds(off, CH)], src_v)
        pltpu.sync_copy(dst_hbm.at[pl.ds(off, CH)], dst_v)
        pltpu.sync_copy(w_hbm.at[pl.ds(off, CH)], w_v)
        pltpu.async_copy(h_hbm.at[src_v], rows_v, sem).wait()
        _scale_rows(rows_v, w_v, CH)
        pltpu.sync_copy(rows_v, acc.at[dst_v], add=True)
        return carry
      lax.fori_loop(0, n_chunks, chunk, 0)

      if tail:
        off = base + n_chunks * CH
        pltpu.sync_copy(src_hbm.at[pl.ds(off, tail)], src_t)
        pltpu.sync_copy(dst_hbm.at[pl.ds(off, tail)], dst_t)
        pltpu.sync_copy(w_hbm.at[pl.ds(off, tail)], w_t)
        pltpu.async_copy(h_hbm.at[src_t], rows_t, sem).wait()
        _scale_rows(rows_t, w_t, tail)
        pltpu.sync_copy(rows_t, acc.at[dst_t], add=True)

      plsc.subcore_barrier()

      @pl.when(sid < NS - 1)
      def _():
        pltpu.sync_copy(
            acc.at[pl.ds(sid * RA, RA)],
            parts_hbm.at[p, cid, pl.ds(sid * RA, RA)])

      @pl.when(sid == NS - 1)
      def _():
        pltpu.sync_copy(
            acc.at[pl.ds((NS - 1) * RA, RB)],
            parts_hbm.at[p, cid, pl.ds((NS - 1) * RA, RB)])

      plsc.subcore_barrier()

  zeros_slab = jnp.zeros((RB, H), dtype=jnp.float32)
  return body(h, src, dst, w, zeros_slab)


def _tc_transform(parts, W, b, prelu_a, fc_W, fc_b):
  P, _, N, H = parts.shape
  BN = 1000 if N % 1000 == 0 else N
  nb = N // BN

  def body(parts_ref, W_ref, b_ref, a_ref, fcW_ref, fcb_ref,
           emb_ref, sacc_ref):
    i = pl.program_id(0)

    @pl.when(i == 0)
    def _():
      sacc_ref[...] = jnp.zeros_like(sacc_ref)

    for p in range(P):
      agg = parts_ref[p, 0] + parts_ref[p, 1]
      fts = lax.dot_general(agg, W_ref[p], (((1,), (1,)), ((), ())),
                            preferred_element_type=jnp.float32)
      x = fts + b_ref[p:p + 1, :]
      a = a_ref[0, p]
      e = jnp.where(x > 0, x, a * x)
      emb_ref[p] = e
      t = jnp.tanh(
          lax.dot_general(e, fcW_ref[...], (((1,), (1,)), ((), ())),
                          preferred_element_type=jnp.float32)
          + fcb_ref[...])
      sacc_ref[p:p + 1, :] += jnp.sum(t, axis=0, keepdims=True)

  emb, sacc = pl.pallas_call(
      body,
      grid=(nb,),
      in_specs=[
          pl.BlockSpec((P, 2, BN, H), lambda i: (0, 0, i, 0)),
          pl.BlockSpec((P, H, H), lambda i: (0, 0, 0)),
          pl.BlockSpec((P, H), lambda i: (0, 0)),
          pl.BlockSpec(memory_space=pltpu.SMEM),
          pl.BlockSpec((H, H), lambda i: (0, 0)),
          pl.BlockSpec((1, H), lambda i: (0, 0)),
      ],
      out_specs=[
          pl.BlockSpec((P, BN, H), lambda i: (0, i, 0)),
          pl.BlockSpec((P, H), lambda i: (0, 0)),
      ],
      out_shape=[
          jax.ShapeDtypeStruct((P, N, H), jnp.float32),
          jax.ShapeDtypeStruct((P, H), jnp.float32),
      ],
  )(parts, W, b, prelu_a.reshape(1, P), fc_W, fc_b.reshape(1, H))
  return emb, sacc


def _tc_combine(emb, sacc, att, n_nodes):
  P, N, H = emb.shape
  BN = 1000 if N % 1000 == 0 else N
  nb = N // BN

  def body(emb_ref, sacc_ref, att_ref, z_ref):
    logits = [
        jnp.sum(att_ref[...] * sacc_ref[p:p + 1, :], axis=1, keepdims=True)
        / n_nodes
        for p in range(P)
    ]
    m = logits[0]
    for p in range(1, P):
      m = jnp.maximum(m, logits[p])
    exps = [jnp.exp(l - m) for l in logits]
    se = exps[0]
    for p in range(1, P):
      se = se + exps[p]
    acc = (exps[0] / se) * emb_ref[0]
    for p in range(1, P):
      acc = acc + (exps[p] / se) * emb_ref[p]
    z_ref[...] = acc

  return pl.pallas_call(
      body,
      grid=(nb,),
      in_specs=[
          pl.BlockSpec((P, BN, H), lambda i: (0, i, 0)),
          pl.BlockSpec((P, H), lambda i: (0, 0)),
          pl.BlockSpec((1, H), lambda i: (0, 0)),
      ],
      out_specs=pl.BlockSpec((BN, H), lambda i: (i, 0)),
      out_shape=jax.ShapeDtypeStruct((N, H), jnp.float32),
  )(emb, sacc, att.reshape(1, H))


def kernel(h, edge_index, edge_weight, W, b, prelu_a, fc_W, fc_b, att):
  N, H = h.shape
  P = edge_index.shape[0]
  dst = edge_index[:, 0, :]
  src = edge_index[:, 1, :]
  parts = _sc_aggregate(h, src, dst, edge_weight)
  emb, sacc = _tc_transform(parts, W, b, prelu_a, fc_W, fc_b)
  return _tc_combine(emb, sacc, att, float(N))
